# baseline (speedup 1.0000x reference)
import functools

import jax
import jax.numpy as jnp
from jax import lax
from jax.experimental import pallas as pl
from jax.experimental.pallas import tpu as pltpu
from jax.experimental.pallas import tpu_sc as plsc

N = 10000
E = 320000
D = 128

NC = 2
NS = 16
NW = NC * NS
CH = 128
ACH0 = 0
ACH1 = 160
TCH = NS * (ACH0 + ACH1)
EPAD = TCH * CH
NP = 10240
RPT = NP // NS
SCH = 8
BLK = 1000


def _make_sc_agg(with_cnt):
    mesh = plsc.VectorSubcoreMesh(core_axis_name="c", subcore_axis_name="s")
    out_type = [jax.ShapeDtypeStruct((NC, NP, D), jnp.float32)]
    if with_cnt:
        out_type.append(jax.ShapeDtypeStruct((NC, NP), jnp.float32))

    def body(table_hbm, srcs_hbm, dsts_hbm, agg_out, *rest):
        if with_cnt:
            cnt_out = rest[0]
            rest = rest[1:]
        (agg_sh, cnt_sh, src_v, dst_v, rows0, rows1, ones_v, zcnt_v,
         gsem0, gsem1, ssem0, ssem1, csem) = rest
        cid = lax.axis_index("c")
        sid = lax.axis_index("s")
        my_nch = jnp.where(cid == 0, ACH0, ACH1)
        my_base = jnp.where(cid == 0, sid * ACH0, NS * ACH0 + sid * ACH1)

        zero16 = jnp.zeros((16,), jnp.float32)
        one16 = jnp.ones((16,), jnp.float32)

        def zrow(r, _):
            def zcol(c2, _):
                rows0[r, pl.ds(c2 * 16, 16)] = zero16
                return 0
            return lax.fori_loop(0, D // 16, zcol, 0, unroll=False)
        lax.fori_loop(0, CH, zrow, 0, unroll=False)

        def ofill(i, _):
            ones_v[pl.ds(i * 16, 16)] = one16
            return 0
        lax.fori_loop(0, CH // 16, ofill, 0, unroll=False)

        def zcfill(i, _):
            zcnt_v[pl.ds(i * 16, 16)] = zero16
            return 0
        lax.fori_loop(0, RPT // 16, zcfill, 0, unroll=False)

        base = sid * RPT
        for k in range(RPT // CH):
            pltpu.sync_copy(rows0, agg_sh.at[pl.ds(base + k * CH, CH)])
        pltpu.sync_copy(zcnt_v, cnt_sh.at[pl.ds(base, RPT)])
        plsc.subcore_barrier()

        def step(jj, _):
            j0 = jj * 2
            j1 = j0 + 1
            pltpu.make_async_copy(table_hbm.at[src_v.at[j0]], rows0, gsem0).wait()
            pltpu.async_copy(rows0, agg_sh.at[dst_v.at[j0]], ssem0, add=True)
            if with_cnt:
                pltpu.async_copy(ones_v, cnt_sh.at[dst_v.at[j0]], csem, add=True)
            pltpu.make_async_copy(table_hbm.at[src_v.at[j1]], rows1, gsem1).wait()

            @pl.when(j0 + 2 < SCH)
            def _():
                pltpu.make_async_copy(rows0, agg_sh.at[dst_v.at[j0]], ssem0).wait()
                pltpu.async_copy(table_hbm.at[src_v.at[j0 + 2]], rows0, gsem0)

            pltpu.async_copy(rows1, agg_sh.at[dst_v.at[j1]], ssem1, add=True)
            if with_cnt:
                pltpu.async_copy(ones_v, cnt_sh.at[dst_v.at[j1]], csem, add=True)

            @pl.when(j1 + 2 < SCH)
            def _():
                pltpu.make_async_copy(rows1, agg_sh.at[dst_v.at[j1]], ssem1).wait()
                pltpu.async_copy(table_hbm.at[src_v.at[j1 + 2]], rows1, gsem1)
            return 0

        def phase_body(ph, _):
            pltpu.sync_copy(
                srcs_hbm.at[pl.ds(my_base + ph * SCH, SCH)], src_v)
            pltpu.sync_copy(
                dsts_hbm.at[pl.ds(my_base + ph * SCH, SCH)], dst_v)
            pltpu.async_copy(table_hbm.at[src_v.at[0]], rows0, gsem0)
            pltpu.async_copy(table_hbm.at[src_v.at[1]], rows1, gsem1)
            lax.fori_loop(0, SCH // 2, step, 0)
            pltpu.make_async_copy(rows0, agg_sh.at[dst_v.at[0]], ssem0).wait()
            pltpu.make_async_copy(rows1, agg_sh.at[dst_v.at[1]], ssem1).wait()
            if with_cnt:
                pltpu.make_async_copy(
                    srcs_hbm.at[pl.ds(0, SCH)], src_v, csem).wait()
            return 0
        lax.fori_loop(0, my_nch // SCH, phase_body, 0)
        plsc.subcore_barrier()

        pltpu.sync_copy(agg_sh.at[pl.ds(base, RPT)],
                        agg_out.at[cid, pl.ds(base, RPT)])
        if with_cnt:
            pltpu.sync_copy(cnt_sh.at[pl.ds(base, RPT)],
                            cnt_out.at[cid, pl.ds(base, RPT)])

    return pl.kernel(
        body,
        out_type=out_type,
        mesh=mesh,
        scratch_types=[
            pltpu.VMEM_SHARED((NP, D), jnp.float32),
            pltpu.VMEM_SHARED((NP,), jnp.float32),
            pltpu.VMEM((SCH, CH), jnp.int32),
            pltpu.VMEM((SCH, CH), jnp.int32),
            pltpu.VMEM((CH, D), jnp.float32),
            pltpu.VMEM((CH, D), jnp.float32),
            pltpu.VMEM((CH,), jnp.float32),
            pltpu.VMEM((RPT,), jnp.float32),
            pltpu.SemaphoreType.DMA,
            pltpu.SemaphoreType.DMA,
            pltpu.SemaphoreType.DMA,
            pltpu.SemaphoreType.DMA,
            pltpu.SemaphoreType.DMA,
        ],
        name="sc_segment_mean" + ("_cnt" if with_cnt else ""),
    )


_sc_agg_cnt = _make_sc_agg(True)
_sc_agg = _make_sc_agg(False)


def _make_dense(act):
    def body(agg_ref, cnt_ref, x_ref, wl_ref, wr_ref, b_ref, o_ref):
        a = agg_ref[0] + agg_ref[1]
        c = cnt_ref[:, 0:1] + cnt_ref[:, 1:2]
        inv = 1.0 / jnp.maximum(c, 1.0)
        mean = a * inv
        z = (jnp.dot(mean, wl_ref[...], preferred_element_type=jnp.float32)
             + jnp.dot(x_ref[...], wr_ref[...], preferred_element_type=jnp.float32)
             + b_ref[...])
        if act == "relu":
            o_ref[...] = jnp.maximum(z, 0.0)
        else:
            m = jnp.max(z, axis=1, keepdims=True)
            e = jnp.exp(z - m)
            o_ref[...] = e / jnp.sum(e, axis=1, keepdims=True)

    return pl.pallas_call(
        body,
        grid=(N // BLK,),
        in_specs=[
            pl.BlockSpec((NC, BLK, D), lambda i: (0, i, 0)),
            pl.BlockSpec((BLK, NC), lambda i: (i, 0)),
            pl.BlockSpec((BLK, D), lambda i: (i, 0)),
            pl.BlockSpec((D, D), lambda i: (0, 0)),
            pl.BlockSpec((D, D), lambda i: (0, 0)),
            pl.BlockSpec((1, D), lambda i: (0, 0)),
        ],
        out_specs=pl.BlockSpec((BLK, D), lambda i: (i, 0)),
        out_shape=jax.ShapeDtypeStruct((N, D), jnp.float32),
        name="sage_dense_" + act,
    )


_dense_relu = _make_dense("relu")
_dense_softmax = _make_dense("softmax")


@jax.jit
def kernel(x, edge_index, W1l, b1l, W1r, W2l, b2l, W2r):
    src = edge_index[0].astype(jnp.int32)
    dst = edge_index[1].astype(jnp.int32)
    npad = EPAD - E
    src = jnp.concatenate([src, jnp.zeros((npad,), jnp.int32)])
    pad_dst = N + (jnp.arange(npad, dtype=jnp.int32) % (NP - N))
    dst = jnp.concatenate([dst, pad_dst])
    srcs = src.reshape(TCH, CH)
    dsts = dst.reshape(TCH, CH)

    agg1, cnt = _sc_agg_cnt(x, srcs, dsts)
    cnt_t = cnt.T
    h = _dense_relu(agg1, cnt_t, x, W1l.T, W1r.T, b1l.reshape(1, D))
    agg2 = _sc_agg(h, srcs, dsts)[0]
    out = _dense_softmax(agg2, cnt_t, h, W2l.T, W2r.T, b2l.reshape(1, D))
    return out


# speedup vs baseline: 1.2974x; 1.2974x over previous
import functools

import jax
import jax.numpy as jnp
from jax import lax
from jax.experimental import pallas as pl
from jax.experimental.pallas import tpu as pltpu
from jax.experimental.pallas import tpu_sc as plsc

N = 10000
E = 320000
D = 128

NC = 2
NS = 16
NW = NC * NS
CH = 128
ACH0 = 156
ACH1 = 4
TCH = NS * (ACH0 + ACH1)
EPAD = TCH * CH
NP = 10240
RPT = NP // NS
SCH = 4
BLK = 1000


def _make_sc_agg(with_cnt):
    mesh = plsc.VectorSubcoreMesh(core_axis_name="c", subcore_axis_name="s")
    out_type = [jax.ShapeDtypeStruct((NC, NP, D), jnp.float32)]
    if with_cnt:
        out_type.append(jax.ShapeDtypeStruct((NC, NP), jnp.float32))

    def body(table_hbm, srcs_hbm, dsts_hbm, agg_out, *rest):
        if with_cnt:
            cnt_out = rest[0]
            rest = rest[1:]
        (agg_sh, cnt_sh, src_v, dst_v, rows0, rows1, ones_v, zcnt_v,
         gsem0, gsem1, ssem0, ssem1, csem) = rest
        cid = lax.axis_index("c")
        sid = lax.axis_index("s")
        my_nch = jnp.where(cid == 0, ACH0, ACH1)
        my_base = jnp.where(cid == 0, sid * ACH0, NS * ACH0 + sid * ACH1)

        zero16 = jnp.zeros((16,), jnp.float32)
        one16 = jnp.ones((16,), jnp.float32)

        def zrow(r, _):
            def zcol(c2, _):
                rows0[r, pl.ds(c2 * 16, 16)] = zero16
                return 0
            return lax.fori_loop(0, D // 16, zcol, 0, unroll=False)
        lax.fori_loop(0, CH, zrow, 0, unroll=False)

        def ofill(i, _):
            ones_v[pl.ds(i * 16, 16)] = one16
            return 0
        lax.fori_loop(0, CH // 16, ofill, 0, unroll=False)

        def zcfill(i, _):
            zcnt_v[pl.ds(i * 16, 16)] = zero16
            return 0
        lax.fori_loop(0, RPT // 16, zcfill, 0, unroll=False)

        base = sid * RPT
        for k in range(RPT // CH):
            pltpu.sync_copy(rows0, agg_sh.at[pl.ds(base + k * CH, CH)])
        pltpu.sync_copy(zcnt_v, cnt_sh.at[pl.ds(base, RPT)])
        plsc.subcore_barrier()

        def step(jj, _):
            j0 = jj * 2
            j1 = j0 + 1
            pltpu.make_async_copy(table_hbm.at[src_v.at[j0]], rows0, gsem0).wait()
            pltpu.async_copy(rows0, agg_sh.at[dst_v.at[j0]], ssem0, add=True)
            if with_cnt:
                pltpu.async_copy(ones_v, cnt_sh.at[dst_v.at[j0]], csem, add=True)
            pltpu.make_async_copy(table_hbm.at[src_v.at[j1]], rows1, gsem1).wait()

            @pl.when(j0 + 2 < SCH)
            def _():
                pltpu.make_async_copy(rows0, agg_sh.at[dst_v.at[j0]], ssem0).wait()
                pltpu.async_copy(table_hbm.at[src_v.at[j0 + 2]], rows0, gsem0)

            pltpu.async_copy(rows1, agg_sh.at[dst_v.at[j1]], ssem1, add=True)
            if with_cnt:
                pltpu.async_copy(ones_v, cnt_sh.at[dst_v.at[j1]], csem, add=True)

            @pl.when(j1 + 2 < SCH)
            def _():
                pltpu.make_async_copy(rows1, agg_sh.at[dst_v.at[j1]], ssem1).wait()
                pltpu.async_copy(table_hbm.at[src_v.at[j1 + 2]], rows1, gsem1)
            return 0

        def phase_body(ph, _):
            pltpu.sync_copy(
                srcs_hbm.at[pl.ds(my_base + ph * SCH, SCH)], src_v)
            pltpu.sync_copy(
                dsts_hbm.at[pl.ds(my_base + ph * SCH, SCH)], dst_v)
            pltpu.async_copy(table_hbm.at[src_v.at[0]], rows0, gsem0)
            pltpu.async_copy(table_hbm.at[src_v.at[1]], rows1, gsem1)
            lax.fori_loop(0, SCH // 2, step, 0)
            pltpu.make_async_copy(rows0, agg_sh.at[dst_v.at[0]], ssem0).wait()
            pltpu.make_async_copy(rows1, agg_sh.at[dst_v.at[1]], ssem1).wait()
            if with_cnt:
                pltpu.make_async_copy(
                    srcs_hbm.at[pl.ds(0, SCH)], src_v, csem).wait()
            return 0
        lax.fori_loop(0, my_nch // SCH, phase_body, 0)
        plsc.subcore_barrier()

        pltpu.sync_copy(agg_sh.at[pl.ds(base, RPT)],
                        agg_out.at[cid, pl.ds(base, RPT)])
        if with_cnt:
            pltpu.sync_copy(cnt_sh.at[pl.ds(base, RPT)],
                            cnt_out.at[cid, pl.ds(base, RPT)])

    return pl.kernel(
        body,
        out_type=out_type,
        mesh=mesh,
        scratch_types=[
            pltpu.VMEM_SHARED((NP, D), jnp.float32),
            pltpu.VMEM_SHARED((NP,), jnp.float32),
            pltpu.VMEM((SCH, CH), jnp.int32),
            pltpu.VMEM((SCH, CH), jnp.int32),
            pltpu.VMEM((CH, D), jnp.float32),
            pltpu.VMEM((CH, D), jnp.float32),
            pltpu.VMEM((CH,), jnp.float32),
            pltpu.VMEM((RPT,), jnp.float32),
            pltpu.SemaphoreType.DMA,
            pltpu.SemaphoreType.DMA,
            pltpu.SemaphoreType.DMA,
            pltpu.SemaphoreType.DMA,
            pltpu.SemaphoreType.DMA,
        ],
        name="sc_segment_mean" + ("_cnt" if with_cnt else ""),
    )


_sc_agg_cnt = _make_sc_agg(True)
_sc_agg = _make_sc_agg(False)


def _make_dense(act):
    def body(agg_ref, cnt_ref, x_ref, wl_ref, wr_ref, b_ref, o_ref):
        a = agg_ref[0] + agg_ref[1]
        c = cnt_ref[:, 0:1] + cnt_ref[:, 1:2]
        inv = 1.0 / jnp.maximum(c, 1.0)
        mean = a * inv
        z = (jnp.dot(mean, wl_ref[...], preferred_element_type=jnp.float32)
             + jnp.dot(x_ref[...], wr_ref[...], preferred_element_type=jnp.float32)
             + b_ref[...])
        if act == "relu":
            o_ref[...] = jnp.maximum(z, 0.0)
        else:
            m = jnp.max(z, axis=1, keepdims=True)
            e = jnp.exp(z - m)
            o_ref[...] = e / jnp.sum(e, axis=1, keepdims=True)

    return pl.pallas_call(
        body,
        grid=(N // BLK,),
        in_specs=[
            pl.BlockSpec((NC, BLK, D), lambda i: (0, i, 0)),
            pl.BlockSpec((BLK, NC), lambda i: (i, 0)),
            pl.BlockSpec((BLK, D), lambda i: (i, 0)),
            pl.BlockSpec((D, D), lambda i: (0, 0)),
            pl.BlockSpec((D, D), lambda i: (0, 0)),
            pl.BlockSpec((1, D), lambda i: (0, 0)),
        ],
        out_specs=pl.BlockSpec((BLK, D), lambda i: (i, 0)),
        out_shape=jax.ShapeDtypeStruct((N, D), jnp.float32),
        name="sage_dense_" + act,
    )


_dense_relu = _make_dense("relu")
_dense_softmax = _make_dense("softmax")


@jax.jit
def kernel(x, edge_index, W1l, b1l, W1r, W2l, b2l, W2r):
    src = edge_index[0].astype(jnp.int32)
    dst = edge_index[1].astype(jnp.int32)
    npad = EPAD - E
    src = jnp.concatenate([src, jnp.zeros((npad,), jnp.int32)])
    pad_dst = N + (jnp.arange(npad, dtype=jnp.int32) % (NP - N))
    dst = jnp.concatenate([dst, pad_dst])
    srcs = src.reshape(TCH, CH)
    dsts = dst.reshape(TCH, CH)

    agg1, cnt = _sc_agg_cnt(x, srcs, dsts)
    cnt_t = cnt.T
    h = _dense_relu(agg1, cnt_t, x, W1l.T, W1r.T, b1l.reshape(1, D))
    agg2 = _sc_agg(h, srcs, dsts)[0]
    out = _dense_softmax(agg2, cnt_t, h, W2l.T, W2r.T, b2l.reshape(1, D))
    return out


# speedup vs baseline: 1.4544x; 1.1210x over previous
import functools

import jax
import jax.numpy as jnp
from jax import lax
from jax.experimental import pallas as pl
from jax.experimental.pallas import tpu as pltpu
from jax.experimental.pallas import tpu_sc as plsc

N = 10000
E = 320000
D = 128

NC = 2
NS = 16
NW = NC * NS
CH = 128
ACH0 = 152
ACH1 = 8
TCH = NS * (ACH0 + ACH1)
EPAD = TCH * CH
NP = 10240
RPT = NP // NS
SCH = 8
BLK = 1000


def _make_sc_agg(with_cnt):
    mesh = plsc.VectorSubcoreMesh(core_axis_name="c", subcore_axis_name="s")
    out_type = [jax.ShapeDtypeStruct((NC, NP, D), jnp.float32)]
    if with_cnt:
        out_type.append(jax.ShapeDtypeStruct((NC, NP), jnp.float32))

    def body(table_hbm, srcs_hbm, dsts_hbm, agg_out, *rest):
        if with_cnt:
            cnt_out = rest[0]
            rest = rest[1:]
        (agg_sh, cnt_sh, src_v, dst_v, rows0, rows1, ones_v, zcnt_v,
         gsem0, gsem1, ssem0, ssem1, csem) = rest
        cid = lax.axis_index("c")
        sid = lax.axis_index("s")
        my_nch = jnp.where(cid == 0, ACH0, ACH1)
        my_base = jnp.where(cid == 0, sid * ACH0, NS * ACH0 + sid * ACH1)

        zero16 = jnp.zeros((16,), jnp.float32)
        one16 = jnp.ones((16,), jnp.float32)

        def zrow(r, _):
            def zcol(c2, _):
                rows0[r, pl.ds(c2 * 16, 16)] = zero16
                return 0
            return lax.fori_loop(0, D // 16, zcol, 0, unroll=False)
        lax.fori_loop(0, CH, zrow, 0, unroll=False)

        def ofill(i, _):
            ones_v[pl.ds(i * 16, 16)] = one16
            return 0
        lax.fori_loop(0, CH // 16, ofill, 0, unroll=False)

        def zcfill(i, _):
            zcnt_v[pl.ds(i * 16, 16)] = zero16
            return 0
        lax.fori_loop(0, RPT // 16, zcfill, 0, unroll=False)

        base = sid * RPT
        for k in range(RPT // CH):
            pltpu.sync_copy(rows0, agg_sh.at[pl.ds(base + k * CH, CH)])
        pltpu.sync_copy(zcnt_v, cnt_sh.at[pl.ds(base, RPT)])
        plsc.subcore_barrier()

        def step(jj, _):
            j0 = jj * 2
            j1 = j0 + 1
            pltpu.make_async_copy(table_hbm.at[src_v.at[j0]], rows0, gsem0).wait()
            pltpu.async_copy(rows0, agg_sh.at[dst_v.at[j0]], ssem0, add=True)
            if with_cnt:
                pltpu.async_copy(ones_v, cnt_sh.at[dst_v.at[j0]], csem, add=True)
            pltpu.make_async_copy(table_hbm.at[src_v.at[j1]], rows1, gsem1).wait()

            @pl.when(j0 + 2 < SCH)
            def _():
                pltpu.make_async_copy(rows0, agg_sh.at[dst_v.at[j0]], ssem0).wait()
                pltpu.async_copy(table_hbm.at[src_v.at[j0 + 2]], rows0, gsem0)

            pltpu.async_copy(rows1, agg_sh.at[dst_v.at[j1]], ssem1, add=True)
            if with_cnt:
                pltpu.async_copy(ones_v, cnt_sh.at[dst_v.at[j1]], csem, add=True)

            @pl.when(j1 + 2 < SCH)
            def _():
                pltpu.make_async_copy(rows1, agg_sh.at[dst_v.at[j1]], ssem1).wait()
                pltpu.async_copy(table_hbm.at[src_v.at[j1 + 2]], rows1, gsem1)
            return 0

        def phase_body(ph, _):
            pltpu.sync_copy(
                srcs_hbm.at[pl.ds(my_base + ph * SCH, SCH)], src_v)
            pltpu.sync_copy(
                dsts_hbm.at[pl.ds(my_base + ph * SCH, SCH)], dst_v)
            pltpu.async_copy(table_hbm.at[src_v.at[0]], rows0, gsem0)
            pltpu.async_copy(table_hbm.at[src_v.at[1]], rows1, gsem1)
            lax.fori_loop(0, SCH // 2, step, 0)
            pltpu.make_async_copy(rows0, agg_sh.at[dst_v.at[0]], ssem0).wait()
            pltpu.make_async_copy(rows1, agg_sh.at[dst_v.at[1]], ssem1).wait()
            if with_cnt:
                pltpu.make_async_copy(
                    srcs_hbm.at[pl.ds(0, SCH)], src_v, csem).wait()
            return 0
        lax.fori_loop(0, my_nch // SCH, phase_body, 0)
        plsc.subcore_barrier()

        pltpu.sync_copy(agg_sh.at[pl.ds(base, RPT)],
                        agg_out.at[cid, pl.ds(base, RPT)])
        if with_cnt:
            pltpu.sync_copy(cnt_sh.at[pl.ds(base, RPT)],
                            cnt_out.at[cid, pl.ds(base, RPT)])

    return pl.kernel(
        body,
        out_type=out_type,
        mesh=mesh,
        scratch_types=[
            pltpu.VMEM_SHARED((NP, D), jnp.float32),
            pltpu.VMEM_SHARED((NP,), jnp.float32),
            pltpu.VMEM((SCH, CH), jnp.int32),
            pltpu.VMEM((SCH, CH), jnp.int32),
            pltpu.VMEM((CH, D), jnp.float32),
            pltpu.VMEM((CH, D), jnp.float32),
            pltpu.VMEM((CH,), jnp.float32),
            pltpu.VMEM((RPT,), jnp.float32),
            pltpu.SemaphoreType.DMA,
            pltpu.SemaphoreType.DMA,
            pltpu.SemaphoreType.DMA,
            pltpu.SemaphoreType.DMA,
            pltpu.SemaphoreType.DMA,
        ],
        name="sc_segment_mean" + ("_cnt" if with_cnt else ""),
    )


_sc_agg_cnt = _make_sc_agg(True)
_sc_agg = _make_sc_agg(False)


def _make_dense(act):
    def body(agg_ref, cnt_ref, x_ref, wl_ref, wr_ref, b_ref, o_ref):
        a = agg_ref[0] + agg_ref[1]
        c = cnt_ref[:, 0:1] + cnt_ref[:, 1:2]
        inv = 1.0 / jnp.maximum(c, 1.0)
        mean = a * inv
        z = (jnp.dot(mean, wl_ref[...], preferred_element_type=jnp.float32)
             + jnp.dot(x_ref[...], wr_ref[...], preferred_element_type=jnp.float32)
             + b_ref[...])
        if act == "relu":
            o_ref[...] = jnp.maximum(z, 0.0)
        else:
            m = jnp.max(z, axis=1, keepdims=True)
            e = jnp.exp(z - m)
            o_ref[...] = e / jnp.sum(e, axis=1, keepdims=True)

    return pl.pallas_call(
        body,
        grid=(N // BLK,),
        in_specs=[
            pl.BlockSpec((NC, BLK, D), lambda i: (0, i, 0)),
            pl.BlockSpec((BLK, NC), lambda i: (i, 0)),
            pl.BlockSpec((BLK, D), lambda i: (i, 0)),
            pl.BlockSpec((D, D), lambda i: (0, 0)),
            pl.BlockSpec((D, D), lambda i: (0, 0)),
            pl.BlockSpec((1, D), lambda i: (0, 0)),
        ],
        out_specs=pl.BlockSpec((BLK, D), lambda i: (i, 0)),
        out_shape=jax.ShapeDtypeStruct((N, D), jnp.float32),
        name="sage_dense_" + act,
    )


_dense_relu = _make_dense("relu")
_dense_softmax = _make_dense("softmax")


@jax.jit
def kernel(x, edge_index, W1l, b1l, W1r, W2l, b2l, W2r):
    src = edge_index[0].astype(jnp.int32)
    dst = edge_index[1].astype(jnp.int32)
    npad = EPAD - E
    src = jnp.concatenate([src, jnp.zeros((npad,), jnp.int32)])
    pad_dst = N + (jnp.arange(npad, dtype=jnp.int32) % (NP - N))
    dst = jnp.concatenate([dst, pad_dst])
    srcs = src.reshape(TCH, CH)
    dsts = dst.reshape(TCH, CH)

    agg1, cnt = _sc_agg_cnt(x, srcs, dsts)
    cnt_t = cnt.T
    h = _dense_relu(agg1, cnt_t, x, W1l.T, W1r.T, b1l.reshape(1, D))
    agg2 = _sc_agg(h, srcs, dsts)[0]
    out = _dense_softmax(agg2, cnt_t, h, W2l.T, W2r.T, b2l.reshape(1, D))
    return out
